# baseline (speedup 1.0000x reference)
import jax
import jax.numpy as jnp
from jax.experimental import pallas as pl
from jax.experimental.pallas import tpu as pltpu

_DS = 4
_MAX_DET = 100
_NMS_THRESH = 0.7


def _bdot(a, b):
    return jnp.dot(a.astype(jnp.bfloat16), b.astype(jnp.bfloat16),
                   preferred_element_type=jnp.float32)


def _conv1_kernel(p_ref, w_ref, b_ref, o_ref):
    p = p_ref[0]
    y = _bdot(p, w_ref[...])
    o_ref[0] = jnp.maximum(y + b_ref[...], 0.0)


def _conv2_kernel(ph_ref, w_ref, b_ref, o_ref):
    s = pl.program_id(1)
    parts = []
    for d in range(3):
        for e in range(3):
            pr, pc = d % 2, e % 2
            u0, v0 = d // 2, e // 2
            blk = ph_ref[0, 2 * pr + pc, pl.ds(16 * s + u0, 16),
                         pl.ds(v0, 96), :]
            parts.append(blk.reshape(1536, 64).astype(jnp.bfloat16))
    patch = jnp.concatenate(parts, axis=1)
    acc = jnp.dot(patch, w_ref[...].astype(jnp.bfloat16),
                  preferred_element_type=jnp.float32)
    o_ref[0] = jnp.maximum(acc + b_ref[...], 0.0)


def _conv3_kernel(x_ref, w_ref, b_ref, wh_ref, bh_ref, o_ref):
    s = pl.program_id(1)
    parts = []
    for d in range(3):
        for e in range(3):
            blk = x_ref[0, pl.ds(16 * s + d, 16), pl.ds(e, 96), :]
            parts.append(blk.reshape(1536, 192).astype(jnp.bfloat16))
    patch = jnp.concatenate(parts, axis=1)
    acc = jnp.dot(patch, w_ref[...].astype(jnp.bfloat16),
                  preferred_element_type=jnp.float32)
    f = jnp.maximum(acc + b_ref[...], 0.0)
    hv = _bdot(f, wh_ref[...]) + bh_ref[...]
    l0 = hv[:, 0:1]
    l1 = hv[:, 1:2]
    m = jnp.maximum(l0, l1)
    e0 = jnp.exp(l0 - m)
    e1 = jnp.exp(l1 - m)
    conf = e1 / (e0 + e1)
    shifted = jnp.concatenate(
        [conf, hv[:, 2:6] * 384.0, jnp.zeros((hv.shape[0], 3), jnp.float32)],
        axis=1)
    o_ref[0] = shifted


def _detect_kernel(conf_ref, loc_ref, o_ref, iou_ref):
    conf = conf_ref[0]
    lane = jax.lax.broadcasted_iota(jnp.int32, (1, 128), 1)
    flat = (jax.lax.broadcasted_iota(jnp.int32, (72, 128), 0) * 128
            + jax.lax.broadcasted_iota(jnp.int32, (72, 128), 1))

    def topk_body(k, c):
        conf_cur, vals, idxs = c
        m = jnp.max(conf_cur)
        cand = jnp.where(conf_cur == m, flat, jnp.int32(2 ** 30))
        imin = jnp.min(cand)
        sel = lane == k
        vals = jnp.where(sel, m, vals)
        idxs = jnp.where(sel, imin, idxs)
        conf_cur = jnp.where(flat == imin, -jnp.inf, conf_cur)
        return conf_cur, vals, idxs

    vals0 = jnp.full((1, 128), -jnp.inf, jnp.float32)
    idxs0 = jnp.zeros((1, 128), jnp.int32)
    _, vals, idxs = jax.lax.fori_loop(0, _MAX_DET, topk_body,
                                      (conf, vals0, idxs0))

    idc = idxs.astype(jnp.float32).T
    yi = jnp.floor(idc / 96.0)
    xi = idc - 96.0 * yi
    xc = xi * float(_DS) + (_DS - 1.0) / 2
    yc = yi * float(_DS) + (_DS - 1.0) / 2
    flat2 = jax.lax.broadcasted_iota(jnp.int32, (128, 9216), 1).astype(
        jnp.float32)
    oh = (flat2 == idc).astype(jnp.float32)
    t = jnp.dot(oh, loc_ref[0], preferred_element_type=jnp.float32)
    bx = xc + t[:, 0:1]
    by = yc + t[:, 1:2]
    x1 = bx - 0.5 * t[:, 2:3]
    x2 = bx + 0.5 * t[:, 2:3]
    y1 = by - 0.5 * t[:, 3:4]
    y2 = by + 0.5 * t[:, 3:4]
    area = jnp.maximum(x2 - x1, 0.0) * jnp.maximum(y2 - y1, 0.0)
    xx1 = jnp.maximum(x1, x1.T)
    yy1 = jnp.maximum(y1, y1.T)
    xx2 = jnp.minimum(x2, x2.T)
    yy2 = jnp.minimum(y2, y2.T)
    inter = jnp.maximum(xx2 - xx1, 0.0) * jnp.maximum(yy2 - yy1, 0.0)
    iou_ref[...] = inter / jnp.maximum(area + area.T - inter, 1e-9)

    keep = (vals >= 0.0).astype(jnp.float32)

    def nms_body(i, keep):
        row = iou_ref[pl.ds(i, 1), :]
        ki = jnp.sum(jnp.where(lane == i, keep, 0.0))
        sup = (ki > 0.0) & (row > _NMS_THRESH) & (lane > i)
        return jnp.where(sup, 0.0, keep)

    keep = jax.lax.fori_loop(0, _MAX_DET, nms_body, keep)
    o_ref[0] = jnp.concatenate(
        [x1.T, y1.T, x2.T, y2.T, vals, keep, jnp.zeros((2, 128))], axis=0)


@jax.jit
def kernel(x, W1, b1, W2, b2, W3, b3, Wc, bc, Wr, br):
    B = x.shape[0]
    f32 = jnp.float32

    xh = jnp.transpose(x, (0, 2, 3, 1))
    xp = jnp.pad(xh, ((0, 0), (0, 1), (0, 1), (0, 0)))
    patches = jnp.concatenate(
        [xp[:, d:d + 384:2, e:e + 384:2, :] for d in range(3)
         for e in range(3)], axis=-1)
    patches = patches.reshape(B, 36864, 27)
    W1p = jnp.transpose(W1, (2, 3, 1, 0)).reshape(27, 64)
    y1 = pl.pallas_call(
        _conv1_kernel,
        grid=(B, 4),
        in_specs=[pl.BlockSpec((1, 9216, 27), lambda b, s: (b, s, 0)),
                  pl.BlockSpec((27, 64), lambda b, s: (0, 0)),
                  pl.BlockSpec((1, 64), lambda b, s: (0, 0))],
        out_specs=pl.BlockSpec((1, 9216, 64), lambda b, s: (b, s, 0)),
        out_shape=jax.ShapeDtypeStruct((B, 36864, 64), f32),
    )(patches, W1p, b1.reshape(1, 64))

    y1 = y1.reshape(B, 192, 192, 64)
    y1p = jnp.pad(y1, ((0, 0), (0, 2), (0, 2), (0, 0)))
    phases = jnp.stack(
        [y1p[:, pr::2, pc::2, :] for pr in range(2) for pc in range(2)],
        axis=1)
    W2r = jnp.transpose(W2, (2, 3, 1, 0)).reshape(576, 192)
    y2 = pl.pallas_call(
        _conv2_kernel,
        grid=(B, 6),
        in_specs=[pl.BlockSpec((1, 4, 97, 97, 64),
                               lambda b, s: (b, 0, 0, 0, 0)),
                  pl.BlockSpec((576, 192), lambda b, s: (0, 0)),
                  pl.BlockSpec((1, 192), lambda b, s: (0, 0))],
        out_specs=pl.BlockSpec((1, 1536, 192), lambda b, s: (b, s, 0)),
        out_shape=jax.ShapeDtypeStruct((B, 9216, 192), f32),
        compiler_params=pltpu.CompilerParams(
            vmem_limit_bytes=100 * 1024 * 1024),
    )(phases, W2r, b2.reshape(1, 192))

    y2 = y2.reshape(B, 96, 96, 192)
    y2p = jnp.pad(y2, ((0, 0), (1, 1), (1, 1), (0, 0)))
    W3r = jnp.transpose(W3, (2, 3, 1, 0)).reshape(1728, 192)
    Wh = jnp.concatenate(
        [Wc[:, :, 0, 0].T, Wr[:, :, 0, 0].T, jnp.zeros((192, 2), f32)],
        axis=1)
    bh = jnp.concatenate([bc, br, jnp.zeros((2,), f32)]).reshape(1, 8)
    hv = pl.pallas_call(
        _conv3_kernel,
        grid=(B, 6),
        in_specs=[pl.BlockSpec((1, 98, 98, 192), lambda b, s: (b, 0, 0, 0)),
                  pl.BlockSpec((1728, 192), lambda b, s: (0, 0)),
                  pl.BlockSpec((1, 192), lambda b, s: (0, 0)),
                  pl.BlockSpec((192, 8), lambda b, s: (0, 0)),
                  pl.BlockSpec((1, 8), lambda b, s: (0, 0))],
        out_specs=pl.BlockSpec((1, 1536, 8), lambda b, s: (b, s, 0)),
        out_shape=jax.ShapeDtypeStruct((B, 9216, 8), f32),
        compiler_params=pltpu.CompilerParams(
            vmem_limit_bytes=100 * 1024 * 1024),
    )(y2p, W3r, b3.reshape(1, 192), Wh, bh)

    conf = hv[:, :, 0].reshape(B, 72, 128)
    loc = hv[:, :, 1:5]
    out = pl.pallas_call(
        _detect_kernel,
        grid=(B,),
        in_specs=[pl.BlockSpec((1, 72, 128), lambda b: (b, 0, 0)),
                  pl.BlockSpec((1, 9216, 4), lambda b: (b, 0, 0))],
        out_specs=pl.BlockSpec((1, 8, 128), lambda b: (b, 0, 0)),
        out_shape=jax.ShapeDtypeStruct((B, 8, 128), f32),
        scratch_shapes=[pltpu.VMEM((128, 128), f32)],
    )(conf, loc)
    dets = jnp.transpose(out[:, 0:5, 0:100], (0, 2, 1))
    keep = out[:, 5, 0:100] > 0.5
    return dets, keep
